# Initial kernel scaffold; baseline (speedup 1.0000x reference)
import jax
import jax.numpy as jnp
from jax.experimental import pallas as pl


def kernel(x_ticker, x_sector, ei_tt_src, ei_tt_dst, ei_ts_src, ei_ts_dst, ei_st_src, ei_st_dst, Wl1_tt, Wr1_tt, b1_tt, Wl1_st, Wr1_st, b1_st, Wl1_ts, Wr1_ts, b1_ts, Wl2_tt, Wr2_tt, b2_tt, Wl2_st, Wr2_st, b2_st, Wl2_ts, Wr2_ts, b2_ts, lin_W, lin_b):
    raise NotImplementedError("write your pallas kernel here")


# speedup vs baseline: 2.9587x; 2.9587x over previous
import functools

import numpy as np
import jax
import jax.numpy as jnp
from jax import lax
from jax.experimental import pallas as pl
from jax.experimental.pallas import tpu as pltpu
from jax.experimental.pallas import tpu_sc as plsc

_N_T, _N_S, _D, _H, _O = 10000, 500, 128, 128, 64
_E_TT, _E_SMALL = 320000, 20000

_NC, _NS = 2, 16
_NW = _NC * _NS
_C = 80
_TT_CH = 128
_TT_PAD = _NW * _TT_CH * _C
_ST_CH = 8
_E_PAD = _NW * _ST_CH * _C
_SB = 32
_ROWS_T = 10240
_ROWS_S = 512


def _leaky(x):
    return jnp.where(x > 0, x, 0.01 * x)


def _tc_mm(x, w, b, splits, blk):
    n = x.shape[0]
    offs = np.cumsum([0] + list(splits))

    def body(x_ref, w_ref, b_ref, *o_refs):
        y = jnp.dot(x_ref[...], w_ref[...], preferred_element_type=jnp.float32)
        y = y + b_ref[...]
        for j, o in enumerate(o_refs):
            o[...] = y[:, offs[j]:offs[j + 1]]

    return pl.pallas_call(
        body,
        grid=(n // blk,),
        in_specs=[
            pl.BlockSpec((blk, x.shape[1]), lambda i: (i, 0)),
            pl.BlockSpec(w.shape, lambda i: (0, 0)),
            pl.BlockSpec((1, w.shape[1]), lambda i: (0, 0)),
        ],
        out_specs=[pl.BlockSpec((blk, c), lambda i: (i, 0)) for c in splits],
        out_shape=tuple(jax.ShapeDtypeStruct((n, c), jnp.float32) for c in splits),
    )(x, w, b)


def _tc_act_mm(acc, r, w, b, splits, blk):
    n = r.shape[0]
    offs = np.cumsum([0] + list(splits))

    def body(a_ref, r_ref, w_ref, b_ref, *o_refs):
        pre = _leaky(a_ref[0] + a_ref[1] + r_ref[...])
        y = jnp.dot(pre, w_ref[...], preferred_element_type=jnp.float32)
        y = y + b_ref[...]
        for j, o in enumerate(o_refs):
            o[...] = y[:, offs[j]:offs[j + 1]]

    return pl.pallas_call(
        body,
        grid=(n // blk,),
        in_specs=[
            pl.BlockSpec((2, blk, acc.shape[2]), lambda i: (0, i, 0)),
            pl.BlockSpec((blk, r.shape[1]), lambda i: (i, 0)),
            pl.BlockSpec(w.shape, lambda i: (0, 0)),
            pl.BlockSpec((1, w.shape[1]), lambda i: (0, 0)),
        ],
        out_specs=[pl.BlockSpec((blk, c), lambda i: (i, 0)) for c in splits],
        out_shape=tuple(jax.ShapeDtypeStruct((n, c), jnp.float32) for c in splits),
    )(acc, r, w, b)


def _sc_segsum(tables, idx_pairs, rel_chunks, rel_acc, acc_rows):
    mesh = plsc.VectorSubcoreMesh(core_axis_name="c", subcore_axis_name="s")
    n_rel = len(tables)
    n_acc = len(acc_rows)

    scratch = [pltpu.VMEM_SHARED((rows, _H), jnp.float32) for rows in acc_rows]
    scratch.append(pltpu.VMEM((_SB, _C), jnp.int32))
    scratch.append(pltpu.VMEM((_SB, _C), jnp.int32))
    scratch.append(pltpu.VMEM((_C, _H), jnp.float32))
    scratch.append(pltpu.SemaphoreType.DMA)

    out_type = tuple(jax.ShapeDtypeStruct((_NC * rows, _H), jnp.float32)
                     for rows in acc_rows)

    @functools.partial(pl.kernel, out_type=out_type, mesh=mesh,
                       scratch_types=scratch)
    def k(*refs):
        tab = refs[:n_rel]
        idx = refs[n_rel:3 * n_rel]
        accs = refs[3 * n_rel:3 * n_rel + n_acc]
        sh = refs[3 * n_rel + n_acc:3 * n_rel + 2 * n_acc]
        p = 3 * n_rel + 2 * n_acc
        stage_s, stage_d, rows_v, sem = refs[p], refs[p + 1], refs[p + 2], refs[p + 3]

        cid = lax.axis_index("c")
        sid = lax.axis_index("s")
        wid = cid * _NS + sid
        zero16 = jnp.zeros((16,), jnp.float32)

        @pl.loop(0, _C)
        def _(i):
            for c in range(_H // 16):
                rows_v[i, pl.ds(c * 16, 16)] = zero16

        for a, rows in enumerate(acc_rows):
            stripe = rows // _NS
            if stripe >= _C:
                for kk in range(stripe // _C):
                    pltpu.sync_copy(rows_v,
                                    sh[a].at[pl.ds(sid * stripe + kk * _C, _C)])
            else:
                pltpu.sync_copy(rows_v.at[pl.ds(0, stripe)],
                                sh[a].at[pl.ds(sid * stripe, stripe)])

        plsc.subcore_barrier()

        for r in range(n_rel):
            nch = rel_chunks[r]
            sb = min(nch, _SB)
            t_h, acc_sh = tab[r], sh[rel_acc[r]]
            ids_h, idd_h = idx[2 * r], idx[2 * r + 1]
            for blk in range(nch // sb):
                base = wid * nch + blk * sb
                pltpu.sync_copy(ids_h.at[pl.ds(base, sb)],
                                stage_s.at[pl.ds(0, sb)])
                pltpu.sync_copy(idd_h.at[pl.ds(base, sb)],
                                stage_d.at[pl.ds(0, sb)])

                @pl.loop(0, sb)
                def _(g, t_h=t_h, acc_sh=acc_sh):
                    pltpu.async_copy(t_h.at[stage_s.at[g]], rows_v, sem).wait()
                    pltpu.sync_copy(rows_v, acc_sh.at[stage_d.at[g]], add=True)

        plsc.subcore_barrier()

        for a, rows in enumerate(acc_rows):
            stripe = rows // _NS
            pltpu.sync_copy(
                sh[a].at[pl.ds(sid * stripe, stripe)],
                accs[a].at[pl.ds(cid * rows + sid * stripe, stripe)])

    flat_idx = []
    for s, dte in idx_pairs:
        flat_idx += [s, dte]
    outs = k(*tables, *flat_idx)
    if not isinstance(outs, (tuple, list)):
        outs = (outs,)
    return [o.reshape(_NC, rows, _H) for o, rows in zip(outs, acc_rows)]


def _pad_idx(src, dst, sink, total):
    pad = total - src.shape[0]
    s = jnp.concatenate([src, jnp.zeros((pad,), jnp.int32)])
    d = jnp.concatenate([dst, jnp.full((pad,), sink, jnp.int32)])
    return s.reshape(-1, _C), d.reshape(-1, _C)


def kernel(x_ticker, x_sector, ei_tt_src, ei_tt_dst, ei_ts_src, ei_ts_dst,
           ei_st_src, ei_st_dst,
           Wl1_tt, Wr1_tt, b1_tt, Wl1_st, Wr1_st, b1_st, Wl1_ts, Wr1_ts, b1_ts,
           Wl2_tt, Wr2_tt, b2_tt, Wl2_st, Wr2_st, b2_st, Wl2_ts, Wr2_ts, b2_ts,
           lin_W, lin_b):
    f32 = jnp.float32
    tt_s, tt_d = _pad_idx(ei_tt_src, ei_tt_dst, _N_T, _TT_PAD)
    st_s, st_d = _pad_idx(ei_st_src, ei_st_dst, _N_T, _E_PAD)
    ts_s, ts_d = _pad_idx(ei_ts_src, ei_ts_dst, _N_S, _E_PAD)

    xs_pad = jnp.pad(x_sector, ((0, 512 - _N_S), (0, 0)))

    w1t = jnp.concatenate([Wl1_tt, Wl1_ts, Wr1_tt + Wr1_st], axis=1)
    b1t = jnp.concatenate([jnp.zeros((2 * _H,), f32), b1_tt + b1_st])[None]
    w1s = jnp.concatenate([Wl1_st, Wr1_ts], axis=1)
    b1s = jnp.concatenate([jnp.zeros((_H,), f32), b1_ts])[None]
    w2t = jnp.concatenate([Wl2_tt, Wr2_tt + Wr2_st], axis=1)
    b2t = jnp.concatenate([jnp.zeros((_H,), f32), b2_tt + b2_st])[None]
    bz = jnp.zeros((1, _H), f32)

    y_tt, y_ts, r_t = _tc_mm(x_ticker, w1t, b1t, [_H, _H, _H], blk=400)
    y_st, r_s = _tc_mm(xs_pad, w1s, b1s, [_H, _H], blk=512)
    acc_t, acc_s = _sc_segsum(
        tables=[y_tt, y_st, y_ts],
        idx_pairs=[(tt_s, tt_d), (st_s, st_d), (ts_s, ts_d)],
        rel_chunks=[_TT_CH, _ST_CH, _ST_CH],
        rel_acc=[0, 0, 1],
        acc_rows=[_ROWS_T, _ROWS_S])

    u_tt, r2 = _tc_act_mm(acc_t, r_t, w2t, b2t, [_H, _H], blk=400)
    (u_st,) = _tc_act_mm(acc_s, r_s, Wl2_st, bz, [_H], blk=512)
    (acc2,) = _sc_segsum(
        tables=[u_tt, u_st],
        idx_pairs=[(tt_s, tt_d), (st_s, st_d)],
        rel_chunks=[_TT_CH, _ST_CH],
        rel_acc=[0, 0],
        acc_rows=[_ROWS_T])

    (out,) = _tc_act_mm(acc2, r2, lin_W, lin_b[None], [_O], blk=400)
    return out
